# baseline (speedup 1.0000x reference)
import functools

import jax
import jax.numpy as jnp
from jax import lax
from jax.experimental import pallas as pl
from jax.experimental.pallas import tpu as pltpu
from jax.experimental.pallas import tpu_sc as plsc

N = 10000
D = 128
H = 128
C = 40

NC = 2
NS = 16
NW = NC * NS
CH = 128

N_ACC = 10112
ROWS_PER_TILE = N_ACC // NS

HALF = N_ACC // 2
NJ = 64
N_AGG = HALF + NJ
AGG_RPT = N_AGG // NS
_BN_RSQRT = 1.0 / (1.0 + 1e-5) ** 0.5


def _sc_mesh():
    return plsc.VectorSubcoreMesh(
        core_axis_name="c", subcore_axis_name="s", num_cores=NC, num_subcores=NS
    )


def _make_deg_kernel(cpw):
    @functools.partial(
        pl.kernel,
        out_type=jax.ShapeDtypeStruct((NC, N_ACC, H), jnp.float32),
        mesh=_sc_mesh(),
        scratch_types=[
            pltpu.VMEM((cpw, CH), jnp.int32),
            pltpu.VMEM((CH, H), jnp.float32),
            pltpu.VMEM_SHARED((N_ACC, H), jnp.float32),
        ],
    )
    def deg_kernel(dstg_hbm, zeros_hbm, out_hbm, dsti_v, ones_v, acc):
        c = lax.axis_index("c")
        s = lax.axis_index("s")
        w = c * NS + s
        r0 = s * ROWS_PER_TILE
        pltpu.sync_copy(zeros_hbm.at[pl.ds(r0, ROWS_PER_TILE)],
                        acc.at[pl.ds(r0, ROWS_PER_TILE)])
        pltpu.sync_copy(dstg_hbm.at[pl.ds(w * cpw, cpw)], dsti_v)

        def fill(i, carry):
            for k in range(8):
                ones_v[i, pl.ds(k * 16, 16)] = jnp.full((16,), 1.0, jnp.float32)
            return carry

        lax.fori_loop(0, CH, fill, 0)
        plsc.subcore_barrier()

        def body(j, carry):
            pltpu.sync_copy(ones_v, acc.at[dsti_v.at[j]], add=True)
            return carry

        lax.fori_loop(0, cpw, body, 0)
        plsc.subcore_barrier()
        pltpu.sync_copy(acc.at[pl.ds(r0, ROWS_PER_TILE)],
                        out_hbm.at[c, pl.ds(r0, ROWS_PER_TILE)])

    return deg_kernel


def _make_agg_kernel(cpw2):
    @functools.partial(
        pl.kernel,
        out_type=jax.ShapeDtypeStruct((NC, N_AGG, H), jnp.float32),
        mesh=_sc_mesh(),
        scratch_types=[
            pltpu.VMEM((cpw2, CH), jnp.int32),
            pltpu.VMEM((cpw2, CH), jnp.int32),
            pltpu.VMEM((2 * CH, H), jnp.float32),
            pltpu.VMEM_SHARED((N_AGG, H), jnp.float32),
            pltpu.SemaphoreType.DMA,
        ],
    )
    def agg_kernel(hp_hbm, srcg_hbm, dstg_hbm, zeros_hbm, out_hbm,
                   srci_v, dsti_v, rows_v, acc, gsem):
        c = lax.axis_index("c")
        s = lax.axis_index("s")
        r0 = s * AGG_RPT
        pltpu.sync_copy(zeros_hbm.at[pl.ds(r0, AGG_RPT)],
                        acc.at[pl.ds(r0, AGG_RPT)])
        pltpu.sync_copy(srcg_hbm.at[pl.ds(s * cpw2, cpw2)], srci_v)
        pltpu.sync_copy(dstg_hbm.at[c, pl.ds(s * cpw2, cpw2)], dsti_v)
        plsc.subcore_barrier()

        ra = rows_v.at[pl.ds(0, CH)]
        rb = rows_v.at[pl.ds(CH, CH)]

        def wait_region(region):
            pltpu.make_async_copy(hp_hbm.at[pl.ds(0, CH)], region, gsem).wait()

        pltpu.async_copy(hp_hbm.at[srci_v.at[0]], ra, gsem)

        def pair(p, carry):
            j = 2 * p
            wait_region(ra)
            pltpu.async_copy(hp_hbm.at[srci_v.at[j + 1]], rb, gsem)
            pltpu.sync_copy(ra, acc.at[dsti_v.at[j]], add=True)
            wait_region(rb)
            jn = jnp.minimum(j + 2, cpw2 - 1)
            pltpu.async_copy(hp_hbm.at[srci_v.at[jn]], ra, gsem)
            pltpu.sync_copy(rb, acc.at[dsti_v.at[j + 1]], add=True)
            return carry

        lax.fori_loop(0, cpw2 // 2, pair, 0)
        wait_region(ra)
        plsc.subcore_barrier()
        pltpu.sync_copy(acc.at[pl.ds(r0, AGG_RPT)],
                        out_hbm.at[c, pl.ds(r0, AGG_RPT)])

    return agg_kernel


_RB = 1000
_GRID = N // _RB


def _t0_body(x_ref, w_ref, d0_ref, d1_ref, a_ref, as_ref, dis_ref):
    deg = 1.0 + d0_ref[...] + d1_ref[...]
    dis = lax.rsqrt(deg)
    a = jnp.dot(x_ref[...], w_ref[...], preferred_element_type=jnp.float32)
    a_ref[...] = a
    as_ref[...] = a * dis
    dis_ref[...] = dis


def _tmid_body(g_ref, ap_ref, dis_ref, b_ref, gp_ref, bt_ref, w_ref,
               an_ref, ans_ref):
    dis = dis_ref[...]
    z = dis * g_ref[...] + (dis * dis) * ap_ref[...] + b_ref[...]
    h = jnp.maximum(z * gp_ref[...] + bt_ref[...], 0.0)
    a = jnp.dot(h, w_ref[...], preferred_element_type=jnp.float32)
    an_ref[...] = a
    ans_ref[...] = a * dis


def _tfin_body(g_ref, ap_ref, dis_ref, b_ref, gp_ref, bt_ref, w_ref,
               bl_ref, o_ref):
    dis = dis_ref[...]
    z = dis * g_ref[...] + (dis * dis) * ap_ref[...] + b_ref[...]
    h = jnp.maximum(z * gp_ref[...] + bt_ref[...], 0.0)
    o_ref[...] = (
        jnp.dot(h, w_ref[...], preferred_element_type=jnp.float32) + bl_ref[...]
    )


def _row_spec(width):
    return pl.BlockSpec((_RB, width), lambda i: (i, 0))


def _full_spec(shape):
    return pl.BlockSpec(shape, lambda i: tuple(0 for _ in shape))


def _t0_call(x, w1t, d0, d1):
    return pl.pallas_call(
        _t0_body,
        grid=(_GRID,),
        in_specs=[_row_spec(D), _full_spec((D, H)), _row_spec(1), _row_spec(1)],
        out_specs=[_row_spec(H), _row_spec(H), _row_spec(1)],
        out_shape=[
            jax.ShapeDtypeStruct((N, H), jnp.float32),
            jax.ShapeDtypeStruct((N, H), jnp.float32),
            jax.ShapeDtypeStruct((N, 1), jnp.float32),
        ],
    )(x, w1t, d0, d1)


def _tmid_call(agg, ap, dis, b, gp, bt, wnt):
    return pl.pallas_call(
        _tmid_body,
        grid=(_GRID,),
        in_specs=[
            _row_spec(H), _row_spec(H), _row_spec(1),
            _full_spec((1, H)), _full_spec((1, H)), _full_spec((1, H)),
            _full_spec((H, H)),
        ],
        out_specs=[_row_spec(H), _row_spec(H)],
        out_shape=[
            jax.ShapeDtypeStruct((N, H), jnp.float32),
            jax.ShapeDtypeStruct((N, H), jnp.float32),
        ],
    )(agg, ap, dis, b, gp, bt, wnt)


def _tfin_call(agg, ap, dis, b, gp, bt, wlt, bl):
    return pl.pallas_call(
        _tfin_body,
        grid=(_GRID,),
        in_specs=[
            _row_spec(H), _row_spec(H), _row_spec(1),
            _full_spec((1, H)), _full_spec((1, H)), _full_spec((1, H)),
            _full_spec((H, C)), _full_spec((1, C)),
        ],
        out_specs=pl.BlockSpec((_RB, C), lambda i: (i, 0)),
        out_shape=jax.ShapeDtypeStruct((N, C), jnp.float32),
    )(agg, ap, dis, b, gp, bt, wlt, bl)


def kernel(x, edge_index, W1, b1, g1, bt1, W2, b2, g2, bt2, W3, b3, g3, bt3,
           Wl, bl):
    E = edge_index.shape[1]
    src = edge_index[0].astype(jnp.int32)
    dst = edge_index[1].astype(jnp.int32)

    cpw = -(-E // (NW * CH))
    cpw = -(-cpw // 8) * 8
    pad = NW * CH * cpw - E
    dst_p = jnp.concatenate([dst, jnp.full((pad,), N, jnp.int32)])
    dstg = dst_p.reshape(NW * cpw, CH)

    cpw2 = -(-E // (NS * CH))
    cpw2 = -(-cpw2 // 8) * 8
    e2 = NS * CH * cpw2
    pad2 = e2 - E
    src2 = jnp.concatenate([src, jnp.zeros((pad2,), jnp.int32)])
    dst2 = jnp.concatenate([dst, jnp.full((pad2,), 2 * HALF, jnp.int32)])
    junk = HALF + (jnp.arange(e2, dtype=jnp.int32) % NJ)
    dst_c0 = jnp.where(dst2 < HALF, dst2, junk)
    dst_c1 = jnp.where(dst2 >= HALF, jnp.minimum(dst2 - HALF, junk), junk)
    srcg2 = src2.reshape(NS * cpw2, CH)
    dstg2 = jnp.stack([dst_c0.reshape(NS * cpw2, CH),
                       dst_c1.reshape(NS * cpw2, CH)])

    zeros_h = jnp.zeros((N_ACC, H), jnp.float32)

    deg_k = _make_deg_kernel(cpw)
    agg_k = _make_agg_kernel(cpw2)

    def agg(hp):
        g = agg_k(hp, srcg2, dstg2, zeros_h)
        return jnp.concatenate([g[0, :HALF], g[1, :HALF]], axis=0)[:N]

    degp = deg_k(dstg, zeros_h)
    d0 = degp[0, :N, 0:1]
    d1 = degp[1, :N, 0:1]

    w1t = W1.T
    w2t = W2.T
    w3t = W3.T
    wlt = Wl.T
    s = jnp.float32(_BN_RSQRT)
    gp1 = (g1 * s).reshape(1, H)
    gp2 = (g2 * s).reshape(1, H)
    gp3 = (g3 * s).reshape(1, H)
    b1r = b1.reshape(1, H)
    b2r = b2.reshape(1, H)
    b3r = b3.reshape(1, H)
    bt1r = bt1.reshape(1, H)
    bt2r = bt2.reshape(1, H)
    bt3r = bt3.reshape(1, H)
    blr = bl.reshape(1, C)

    a1, a1s, dis = _t0_call(x, w1t, d0, d1)
    a2, a2s = _tmid_call(agg(a1s), a1, dis, b1r, gp1, bt1r, w2t)
    a3, a3s = _tmid_call(agg(a2s), a2, dis, b2r, gp2, bt2r, w3t)
    return _tfin_call(agg(a3s), a3, dis, b3r, gp3, bt3r, wlt, blr)


# speedup vs baseline: 1.6028x; 1.6028x over previous
import functools

import jax
import jax.numpy as jnp
from jax import lax
from jax.experimental import pallas as pl
from jax.experimental.pallas import tpu as pltpu
from jax.experimental.pallas import tpu_sc as plsc

N = 10000
D = 128
H = 128
C = 40

NC = 2
NS = 16
NW = NC * NS
CH = 128

N_ACC = 10112
ROWS_PER_TILE = N_ACC // NS
_BN_RSQRT = 1.0 / (1.0 + 1e-5) ** 0.5


def _sc_mesh():
    return plsc.VectorSubcoreMesh(
        core_axis_name="c", subcore_axis_name="s", num_cores=NC, num_subcores=NS
    )


def _make_deg_kernel(cpw):
    @functools.partial(
        pl.kernel,
        out_type=jax.ShapeDtypeStruct((NC, N_ACC, H), jnp.float32),
        mesh=_sc_mesh(),
        scratch_types=[
            pltpu.VMEM((cpw, CH), jnp.int32),
            pltpu.VMEM((CH, H), jnp.float32),
            pltpu.VMEM_SHARED((N_ACC, H), jnp.float32),
        ],
    )
    def deg_kernel(dstg_hbm, zeros_hbm, out_hbm, dsti_v, ones_v, acc):
        c = lax.axis_index("c")
        s = lax.axis_index("s")
        w = c * NS + s
        r0 = s * ROWS_PER_TILE
        pltpu.sync_copy(zeros_hbm.at[pl.ds(r0, ROWS_PER_TILE)],
                        acc.at[pl.ds(r0, ROWS_PER_TILE)])
        pltpu.sync_copy(dstg_hbm.at[pl.ds(w * cpw, cpw)], dsti_v)

        def fill(i, carry):
            for k in range(8):
                ones_v[i, pl.ds(k * 16, 16)] = jnp.full((16,), 1.0, jnp.float32)
            return carry

        lax.fori_loop(0, CH, fill, 0)
        plsc.subcore_barrier()

        def body(j, carry):
            pltpu.sync_copy(ones_v, acc.at[dsti_v.at[j]], add=True)
            return carry

        lax.fori_loop(0, cpw, body, 0)
        plsc.subcore_barrier()
        pltpu.sync_copy(acc.at[pl.ds(r0, ROWS_PER_TILE)],
                        out_hbm.at[c, pl.ds(r0, ROWS_PER_TILE)])

    return deg_kernel


def _make_agg_kernel(cpw0, cpw1):
    cpwm = max(cpw0, cpw1)

    @functools.partial(
        pl.kernel,
        out_type=jax.ShapeDtypeStruct((NC, N_ACC, H), jnp.float32),
        mesh=_sc_mesh(),
        scratch_types=[
            pltpu.VMEM((cpwm, CH), jnp.int32),
            pltpu.VMEM((cpwm, CH), jnp.int32),
            pltpu.VMEM((CH, H), jnp.float32),
            pltpu.VMEM_SHARED((N_ACC, H), jnp.float32),
            pltpu.SemaphoreType.DMA,
        ],
    )
    def agg_kernel(hp_hbm, srcg_hbm, dstg_hbm, zeros_hbm, out_hbm,
                   srci_v, dsti_v, rows_v, acc, gsem):
        c = lax.axis_index("c")
        s = lax.axis_index("s")
        row0 = jnp.where(c == 0, s * cpw0, NS * cpw0 + s * cpw1)
        nb = jnp.where(c == 0, cpw0, cpw1)
        r0 = s * ROWS_PER_TILE
        pltpu.sync_copy(zeros_hbm.at[pl.ds(r0, ROWS_PER_TILE)],
                        acc.at[pl.ds(r0, ROWS_PER_TILE)])
        pltpu.sync_copy(srcg_hbm.at[pl.ds(row0, cpwm)], srci_v)
        pltpu.sync_copy(dstg_hbm.at[pl.ds(row0, cpwm)], dsti_v)
        plsc.subcore_barrier()

        def body(j, carry):
            pltpu.async_copy(hp_hbm.at[srci_v.at[j]], rows_v, gsem).wait()
            pltpu.sync_copy(rows_v, acc.at[dsti_v.at[j]], add=True)
            return carry

        lax.fori_loop(0, nb, body, 0)
        plsc.subcore_barrier()
        pltpu.sync_copy(acc.at[pl.ds(r0, ROWS_PER_TILE)],
                        out_hbm.at[c, pl.ds(r0, ROWS_PER_TILE)])

    return agg_kernel


_RB = 1000
_GRID = N // _RB


def _t0_body(x_ref, w_ref, d0_ref, d1_ref, a_ref, as_ref, dis_ref):
    deg = 1.0 + d0_ref[...] + d1_ref[...]
    dis = lax.rsqrt(deg)
    a = jnp.dot(x_ref[...], w_ref[...], preferred_element_type=jnp.float32)
    a_ref[...] = a
    as_ref[...] = a * dis
    dis_ref[...] = dis


def _tmid_body(g0_ref, g1_ref, ap_ref, dis_ref, b_ref, gp_ref, bt_ref, w_ref,
               an_ref, ans_ref):
    dis = dis_ref[...]
    z = dis * (g0_ref[0] + g1_ref[0]) + (dis * dis) * ap_ref[...] + b_ref[...]
    h = jnp.maximum(z * gp_ref[...] + bt_ref[...], 0.0)
    a = jnp.dot(h, w_ref[...], preferred_element_type=jnp.float32)
    an_ref[...] = a
    ans_ref[...] = a * dis


def _tfin_body(g0_ref, g1_ref, ap_ref, dis_ref, b_ref, gp_ref, bt_ref, w_ref,
               bl_ref, o_ref):
    dis = dis_ref[...]
    z = dis * (g0_ref[0] + g1_ref[0]) + (dis * dis) * ap_ref[...] + b_ref[...]
    h = jnp.maximum(z * gp_ref[...] + bt_ref[...], 0.0)
    o_ref[...] = (
        jnp.dot(h, w_ref[...], preferred_element_type=jnp.float32) + bl_ref[...]
    )


def _row_spec(width):
    return pl.BlockSpec((_RB, width), lambda i: (i, 0))


def _full_spec(shape):
    return pl.BlockSpec(shape, lambda i: tuple(0 for _ in shape))


def _agg_spec(core):
    return pl.BlockSpec((1, _RB, H), lambda i, c=core: (c, i, 0))


def _t0_call(x, w1t, d0, d1):
    return pl.pallas_call(
        _t0_body,
        grid=(_GRID,),
        in_specs=[_row_spec(D), _full_spec((D, H)), _row_spec(1), _row_spec(1)],
        out_specs=[_row_spec(H), _row_spec(H), _row_spec(1)],
        out_shape=[
            jax.ShapeDtypeStruct((N, H), jnp.float32),
            jax.ShapeDtypeStruct((N, H), jnp.float32),
            jax.ShapeDtypeStruct((N, 1), jnp.float32),
        ],
    )(x, w1t, d0, d1)


def _tmid_call(agg, ap, dis, b, gp, bt, wnt):
    return pl.pallas_call(
        _tmid_body,
        grid=(_GRID,),
        in_specs=[
            _agg_spec(0), _agg_spec(1), _row_spec(H), _row_spec(1),
            _full_spec((1, H)), _full_spec((1, H)), _full_spec((1, H)),
            _full_spec((H, H)),
        ],
        out_specs=[_row_spec(H), _row_spec(H)],
        out_shape=[
            jax.ShapeDtypeStruct((N, H), jnp.float32),
            jax.ShapeDtypeStruct((N, H), jnp.float32),
        ],
    )(agg, agg, ap, dis, b, gp, bt, wnt)


def _tfin_call(agg, ap, dis, b, gp, bt, wlt, bl):
    return pl.pallas_call(
        _tfin_body,
        grid=(_GRID,),
        in_specs=[
            _agg_spec(0), _agg_spec(1), _row_spec(H), _row_spec(1),
            _full_spec((1, H)), _full_spec((1, H)), _full_spec((1, H)),
            _full_spec((H, C)), _full_spec((1, C)),
        ],
        out_specs=pl.BlockSpec((_RB, C), lambda i: (i, 0)),
        out_shape=jax.ShapeDtypeStruct((N, C), jnp.float32),
    )(agg, agg, ap, dis, b, gp, bt, wlt, bl)


def kernel(x, edge_index, W1, b1, g1, bt1, W2, b2, g2, bt2, W3, b3, g3, bt3,
           Wl, bl):
    E = edge_index.shape[1]
    src = edge_index[0].astype(jnp.int32)
    dst = edge_index[1].astype(jnp.int32)

    cpw = -(-E // (NW * CH))
    cpw = -(-cpw // 8) * 8
    pad = NW * CH * cpw - E
    dstg = jnp.concatenate([dst, jnp.full((pad,), N, jnp.int32)]).reshape(
        NW * cpw, CH)

    total = -(-E // (NS * CH))
    total = -(-total // 16) * 16
    cpw0 = total * 2 // 5
    cpw0 = -(-cpw0 // 8) * 8
    cpw1 = total - cpw0
    rows_total = NS * total + max(cpw0, cpw1)
    pad2 = rows_total * CH - E
    srcg_a = jnp.concatenate([src, jnp.zeros((pad2,), jnp.int32)]).reshape(
        rows_total, CH)
    dstg_a = jnp.concatenate([dst, jnp.full((pad2,), N, jnp.int32)]).reshape(
        rows_total, CH)

    zeros_h = jnp.zeros((N_ACC, H), jnp.float32)

    deg_k = _make_deg_kernel(cpw)
    agg_k = _make_agg_kernel(cpw0, cpw1)

    degp = deg_k(dstg, zeros_h)
    d0 = degp[0, :N, 0:1]
    d1 = degp[1, :N, 0:1]

    w1t = W1.T
    w2t = W2.T
    w3t = W3.T
    wlt = Wl.T
    s = jnp.float32(_BN_RSQRT)
    gp1 = (g1 * s).reshape(1, H)
    gp2 = (g2 * s).reshape(1, H)
    gp3 = (g3 * s).reshape(1, H)
    b1r = b1.reshape(1, H)
    b2r = b2.reshape(1, H)
    b3r = b3.reshape(1, H)
    bt1r = bt1.reshape(1, H)
    bt2r = bt2.reshape(1, H)
    bt3r = bt3.reshape(1, H)
    blr = bl.reshape(1, C)

    a1, a1s, dis = _t0_call(x, w1t, d0, d1)
    agg1 = agg_k(a1s, srcg_a, dstg_a, zeros_h)
    a2, a2s = _tmid_call(agg1, a1, dis, b1r, gp1, bt1r, w2t)
    agg2 = agg_k(a2s, srcg_a, dstg_a, zeros_h)
    a3, a3s = _tmid_call(agg2, a2, dis, b2r, gp2, bt2r, w3t)
    agg3 = agg_k(a3s, srcg_a, dstg_a, zeros_h)
    return _tfin_call(agg3, a3, dis, b3r, gp3, bt3r, wlt, blr)


# speedup vs baseline: 1.7530x; 1.0937x over previous
import functools

import jax
import jax.numpy as jnp
from jax import lax
from jax.experimental import pallas as pl
from jax.experimental.pallas import tpu as pltpu
from jax.experimental.pallas import tpu_sc as plsc

N = 10000
D = 128
H = 128
C = 40

NC = 2
NS = 16
NW = NC * NS
CH = 128

N_ACC = 10112
ROWS_PER_TILE = N_ACC // NS
_BN_RSQRT = 1.0 / (1.0 + 1e-5) ** 0.5


def _sc_mesh():
    return plsc.VectorSubcoreMesh(
        core_axis_name="c", subcore_axis_name="s", num_cores=NC, num_subcores=NS
    )


def _make_deg_kernel(cpw):
    @functools.partial(
        pl.kernel,
        out_type=jax.ShapeDtypeStruct((NC, N_ACC, H), jnp.float32),
        mesh=_sc_mesh(),
        scratch_types=[
            pltpu.VMEM((cpw, CH), jnp.int32),
            pltpu.VMEM((CH, H), jnp.float32),
            pltpu.VMEM_SHARED((N_ACC, H), jnp.float32),
        ],
    )
    def deg_kernel(dstg_hbm, zeros_hbm, out_hbm, dsti_v, ones_v, acc):
        c = lax.axis_index("c")
        s = lax.axis_index("s")
        w = c * NS + s
        r0 = s * ROWS_PER_TILE
        pltpu.sync_copy(zeros_hbm.at[pl.ds(r0, ROWS_PER_TILE)],
                        acc.at[pl.ds(r0, ROWS_PER_TILE)])
        pltpu.sync_copy(dstg_hbm.at[pl.ds(w * cpw, cpw)], dsti_v)

        def fill(i, carry):
            for k in range(8):
                ones_v[i, pl.ds(k * 16, 16)] = jnp.full((16,), 1.0, jnp.float32)
            return carry

        lax.fori_loop(0, CH, fill, 0)
        plsc.subcore_barrier()

        def body(j, carry):
            pltpu.sync_copy(ones_v, acc.at[dsti_v.at[j]], add=True)
            return carry

        lax.fori_loop(0, cpw, body, 0)
        plsc.subcore_barrier()
        pltpu.sync_copy(acc.at[pl.ds(r0, ROWS_PER_TILE)],
                        out_hbm.at[c, pl.ds(r0, ROWS_PER_TILE)])

    return deg_kernel


def _make_agg_kernel(cpw0, cpw1):
    cpwm = max(cpw0, cpw1)

    @functools.partial(
        pl.kernel,
        out_type=jax.ShapeDtypeStruct((NC, N_ACC, H), jnp.float32),
        mesh=_sc_mesh(),
        scratch_types=[
            pltpu.VMEM((cpwm, CH), jnp.int32),
            pltpu.VMEM((cpwm, CH), jnp.int32),
            pltpu.VMEM((CH, H), jnp.float32),
            pltpu.VMEM_SHARED((N_ACC, H), jnp.float32),
            pltpu.SemaphoreType.DMA,
        ],
    )
    def agg_kernel(hp_hbm, srcg_hbm, dstg_hbm, zeros_hbm, out_hbm,
                   srci_v, dsti_v, rows_v, acc, gsem):
        c = lax.axis_index("c")
        s = lax.axis_index("s")
        row0 = jnp.where(c == 0, s * cpw0, NS * cpw0 + s * cpw1)
        nb = jnp.where(c == 0, cpw0, cpw1)
        r0 = s * ROWS_PER_TILE
        pltpu.sync_copy(zeros_hbm.at[pl.ds(r0, ROWS_PER_TILE)],
                        acc.at[pl.ds(r0, ROWS_PER_TILE)])
        pltpu.sync_copy(srcg_hbm.at[pl.ds(row0, cpwm)], srci_v)
        pltpu.sync_copy(dstg_hbm.at[pl.ds(row0, cpwm)], dsti_v)
        plsc.subcore_barrier()

        def body(j, carry):
            pltpu.async_copy(hp_hbm.at[srci_v.at[j]], rows_v, gsem).wait()
            pltpu.sync_copy(rows_v, acc.at[dsti_v.at[j]], add=True)
            return carry

        lax.fori_loop(0, nb, body, 0)
        plsc.subcore_barrier()
        pltpu.sync_copy(acc.at[pl.ds(r0, ROWS_PER_TILE)],
                        out_hbm.at[c, pl.ds(r0, ROWS_PER_TILE)])

    return agg_kernel


_RB = 1000
_GRID = N // _RB


def _t0_body(x_ref, w_ref, d0_ref, d1_ref, a_ref, as_ref, dis_ref):
    deg = 1.0 + d0_ref[...] + d1_ref[...]
    dis = lax.rsqrt(deg)
    a = jnp.dot(x_ref[...], w_ref[...], preferred_element_type=jnp.float32)
    a_ref[...] = a
    as_ref[...] = a * dis
    dis_ref[...] = dis


def _tmid_body(g0_ref, g1_ref, ap_ref, dis_ref, b_ref, gp_ref, bt_ref, w_ref,
               an_ref, ans_ref):
    dis = dis_ref[...]
    z = dis * (g0_ref[0] + g1_ref[0]) + (dis * dis) * ap_ref[...] + b_ref[...]
    h = jnp.maximum(z * gp_ref[...] + bt_ref[...], 0.0)
    a = jnp.dot(h, w_ref[...], preferred_element_type=jnp.float32)
    an_ref[...] = a
    ans_ref[...] = a * dis


def _tfin_body(g0_ref, g1_ref, ap_ref, dis_ref, b_ref, gp_ref, bt_ref, w_ref,
               bl_ref, o_ref):
    dis = dis_ref[...]
    z = dis * (g0_ref[0] + g1_ref[0]) + (dis * dis) * ap_ref[...] + b_ref[...]
    h = jnp.maximum(z * gp_ref[...] + bt_ref[...], 0.0)
    o_ref[...] = (
        jnp.dot(h, w_ref[...], preferred_element_type=jnp.float32) + bl_ref[...]
    )


def _row_spec(width):
    return pl.BlockSpec((_RB, width), lambda i: (i, 0))


def _full_spec(shape):
    return pl.BlockSpec(shape, lambda i: tuple(0 for _ in shape))


def _agg_spec(core):
    return pl.BlockSpec((1, _RB, H), lambda i, c=core: (c, i, 0))


def _t0_call(x, w1t, d0, d1):
    return pl.pallas_call(
        _t0_body,
        grid=(_GRID,),
        in_specs=[_row_spec(D), _full_spec((D, H)), _row_spec(1), _row_spec(1)],
        out_specs=[_row_spec(H), _row_spec(H), _row_spec(1)],
        out_shape=[
            jax.ShapeDtypeStruct((N, H), jnp.float32),
            jax.ShapeDtypeStruct((N, H), jnp.float32),
            jax.ShapeDtypeStruct((N, 1), jnp.float32),
        ],
    )(x, w1t, d0, d1)


def _tmid_call(agg, ap, dis, b, gp, bt, wnt):
    return pl.pallas_call(
        _tmid_body,
        grid=(_GRID,),
        in_specs=[
            _agg_spec(0), _agg_spec(1), _row_spec(H), _row_spec(1),
            _full_spec((1, H)), _full_spec((1, H)), _full_spec((1, H)),
            _full_spec((H, H)),
        ],
        out_specs=[_row_spec(H), _row_spec(H)],
        out_shape=[
            jax.ShapeDtypeStruct((N, H), jnp.float32),
            jax.ShapeDtypeStruct((N, H), jnp.float32),
        ],
    )(agg, agg, ap, dis, b, gp, bt, wnt)


def _tfin_call(agg, ap, dis, b, gp, bt, wlt, bl):
    return pl.pallas_call(
        _tfin_body,
        grid=(_GRID,),
        in_specs=[
            _agg_spec(0), _agg_spec(1), _row_spec(H), _row_spec(1),
            _full_spec((1, H)), _full_spec((1, H)), _full_spec((1, H)),
            _full_spec((H, C)), _full_spec((1, C)),
        ],
        out_specs=pl.BlockSpec((_RB, C), lambda i: (i, 0)),
        out_shape=jax.ShapeDtypeStruct((N, C), jnp.float32),
    )(agg, agg, ap, dis, b, gp, bt, wlt, bl)


def kernel(x, edge_index, W1, b1, g1, bt1, W2, b2, g2, bt2, W3, b3, g3, bt3,
           Wl, bl):
    E = edge_index.shape[1]
    src = edge_index[0].astype(jnp.int32)
    dst = edge_index[1].astype(jnp.int32)

    cpw = -(-E // (NW * CH))
    cpw = -(-cpw // 8) * 8
    pad = NW * CH * cpw - E
    dstg = jnp.concatenate([dst, jnp.full((pad,), N, jnp.int32)]).reshape(
        NW * cpw, CH)

    total = -(-E // (NS * CH))
    total = -(-total // 16) * 16
    cpw0 = total * 3 // 5
    cpw0 = -(-cpw0 // 8) * 8
    cpw1 = total - cpw0
    rows_total = NS * total + max(cpw0, cpw1)
    pad2 = rows_total * CH - E
    srcg_a = jnp.concatenate([src, jnp.zeros((pad2,), jnp.int32)]).reshape(
        rows_total, CH)
    dstg_a = jnp.concatenate([dst, jnp.full((pad2,), N, jnp.int32)]).reshape(
        rows_total, CH)

    zeros_h = jnp.zeros((N_ACC, H), jnp.float32)

    deg_k = _make_deg_kernel(cpw)
    agg_k = _make_agg_kernel(cpw0, cpw1)

    degp = deg_k(dstg, zeros_h)
    d0 = degp[0, :N, 0:1]
    d1 = degp[1, :N, 0:1]

    w1t = W1.T
    w2t = W2.T
    w3t = W3.T
    wlt = Wl.T
    s = jnp.float32(_BN_RSQRT)
    gp1 = (g1 * s).reshape(1, H)
    gp2 = (g2 * s).reshape(1, H)
    gp3 = (g3 * s).reshape(1, H)
    b1r = b1.reshape(1, H)
    b2r = b2.reshape(1, H)
    b3r = b3.reshape(1, H)
    bt1r = bt1.reshape(1, H)
    bt2r = bt2.reshape(1, H)
    bt3r = bt3.reshape(1, H)
    blr = bl.reshape(1, C)

    a1, a1s, dis = _t0_call(x, w1t, d0, d1)
    agg1 = agg_k(a1s, srcg_a, dstg_a, zeros_h)
    a2, a2s = _tmid_call(agg1, a1, dis, b1r, gp1, bt1r, w2t)
    agg2 = agg_k(a2s, srcg_a, dstg_a, zeros_h)
    a3, a3s = _tmid_call(agg2, a2, dis, b2r, gp2, bt2r, w3t)
    agg3 = agg_k(a3s, srcg_a, dstg_a, zeros_h)
    return _tfin_call(agg3, a3, dis, b3r, gp3, bt3r, wlt, blr)
